# Initial kernel scaffold; baseline (speedup 1.0000x reference)
import jax
import jax.numpy as jnp
from jax.experimental import pallas as pl


def kernel(x, sign1, sign2, sign3, hash1, hash2, hash3):
    raise NotImplementedError("write your pallas kernel here")


# speedup vs baseline: 2.6760x; 2.6760x over previous
import functools

import jax
import jax.numpy as jnp
from jax.experimental import pallas as pl
from jax.experimental.pallas import tpu as pltpu

B = 4096
D = 2048
S = 4096

B_BLK = 512
S_BLK = 1024


def _body(x_ref, h1_ref, h2_ref, h3_ref, s1_ref, s2_ref, s3_ref, out_ref,
          m1_ref, m2_ref, m3_ref):
    s_idx = pl.program_id(0)
    b_idx = pl.program_id(1)

    @pl.when(b_idx == 0)
    def _build_onehots():
        col = jax.lax.broadcasted_iota(jnp.int32, (D, S_BLK), 1) + s_idx * S_BLK
        zero = jnp.zeros((D, S_BLK), dtype=jnp.float32)
        m1_ref[...] = jnp.where(h1_ref[...] == col, s1_ref[...], zero).astype(jnp.bfloat16)
        m2_ref[...] = jnp.where(h2_ref[...] == col, s2_ref[...], zero).astype(jnp.bfloat16)
        m3_ref[...] = jnp.where(h3_ref[...] == col, s3_ref[...], zero).astype(jnp.bfloat16)

    xb = x_ref[...]
    a1 = jnp.dot(xb, m1_ref[...], preferred_element_type=jnp.float32)
    a2 = jnp.dot(xb, m2_ref[...], preferred_element_type=jnp.float32)
    a3 = jnp.dot(xb, m3_ref[...], preferred_element_type=jnp.float32)
    out_ref[...] = a1 * a2 * a3


@functools.partial(jax.jit, static_argnums=())
def kernel(x, sign1, sign2, sign3, hash1, hash2, hash3):
    x16 = x.astype(jnp.bfloat16)
    h1 = hash1.reshape(D, 1)
    h2 = hash2.reshape(D, 1)
    h3 = hash3.reshape(D, 1)
    s1 = sign1.reshape(D, 1)
    s2 = sign2.reshape(D, 1)
    s3 = sign3.reshape(D, 1)

    full = lambda: pl.BlockSpec((D, 1), lambda s, b: (0, 0))
    return pl.pallas_call(
        _body,
        grid=(S // S_BLK, B // B_BLK),
        in_specs=[
            pl.BlockSpec((B_BLK, D), lambda s, b: (b, 0)),
            full(), full(), full(), full(), full(), full(),
        ],
        out_specs=pl.BlockSpec((B_BLK, S_BLK), lambda s, b: (b, s)),
        out_shape=jax.ShapeDtypeStruct((B, S), jnp.float32),
        scratch_shapes=[
            pltpu.VMEM((D, S_BLK), jnp.bfloat16),
            pltpu.VMEM((D, S_BLK), jnp.bfloat16),
            pltpu.VMEM((D, S_BLK), jnp.bfloat16),
        ],
        compiler_params=pltpu.CompilerParams(
            dimension_semantics=("arbitrary", "arbitrary"),
        ),
    )(x16, h1, h2, h3, s1, s2, s3)
